# Initial kernel scaffold; baseline (speedup 1.0000x reference)
import functools
import math

import jax
import jax.numpy as jnp
from jax import lax
from jax.experimental import pallas as pl
from jax.experimental.pallas import tpu as pltpu
from jax.experimental.pallas import tpu_sc as plsc

D_MODEL = 128
SQRT_D = math.sqrt(D_MODEL)

_NC = 2
_NS = 16
_NW = _NC * _NS
_CHUNK = 128


def _scaled_table(table):
    V, D = table.shape
    blk = 2000

    def body(t_ref, o_ref):
        o_ref[...] = t_ref[...] + SQRT_D

    return pl.pallas_call(
        body,
        out_shape=jax.ShapeDtypeStruct((V, D), table.dtype),
        grid=(V // blk,),
        in_specs=[pl.BlockSpec((blk, D), lambda i: (i, 0))],
        out_specs=pl.BlockSpec((blk, D), lambda i: (i, 0)),
    )(table)


@functools.cache
def _make_gather(B, D):
    b_per_w = B // _NW
    n_chunks = b_per_w // _CHUNK
    mesh = plsc.VectorSubcoreMesh(core_axis_name="c", subcore_axis_name="s")

    @functools.partial(
        pl.kernel,
        mesh=mesh,
        out_type=jax.ShapeDtypeStruct((B, D), jnp.float32),
        scratch_types=[
            pltpu.VMEM((b_per_w,), jnp.int32),
            pltpu.VMEM((2, _CHUNK, D), jnp.float32),
            pltpu.SemaphoreType.DMA,
            pltpu.SemaphoreType.DMA,
        ],
    )
    def k(table_hbm, idx_hbm, out_hbm, idx_v, rows_v, gsem, osem):
        wid = lax.axis_index("s") * _NC + lax.axis_index("c")
        base = pl.multiple_of(wid * b_per_w, 8)
        pltpu.sync_copy(idx_hbm.at[pl.ds(base, b_per_w)], idx_v)

        def gather(c, slot):
            off = pl.multiple_of(c * _CHUNK, 8)
            return pltpu.async_copy(
                table_hbm.at[idx_v.at[pl.ds(off, _CHUNK)]],
                rows_v.at[slot], gsem)

        def put(c, slot):
            off = pl.multiple_of(base + c * _CHUNK, 8)
            return pltpu.async_copy(
                rows_v.at[slot], out_hbm.at[pl.ds(off, _CHUNK)], osem)

        gather(0, 0).wait()

        def body(c, carry):
            slot = lax.rem(c, 2)
            gather(c + 1, 1 - slot).wait()
            put(c, slot).wait()
            return carry

        lax.fori_loop(0, n_chunks - 1, body, 0)
        put(n_chunks - 1, lax.rem(n_chunks - 1, 2)).wait()

    return k


def kernel(x, table):
    B = x.size
    flat = x.reshape(B)
    table2 = _scaled_table(table)
    out = _make_gather(B, D_MODEL)(table2, flat)
    return out.reshape(x.shape + (D_MODEL,))


# speedup vs baseline: 6.6802x; 6.6802x over previous
import functools
import math

import jax
import jax.numpy as jnp
from jax import lax
from jax.experimental import pallas as pl
from jax.experimental.pallas import tpu as pltpu
from jax.experimental.pallas import tpu_sc as plsc

D_MODEL = 128
SQRT_D = math.sqrt(D_MODEL)

_NC = 2
_NS = 16
_NW = _NC * _NS
_CHUNK = 128


def _scaled_table(table):
    V, D = table.shape
    blk = 2000

    def body(t_ref, o_ref):
        o_ref[...] = t_ref[...] + SQRT_D

    return pl.pallas_call(
        body,
        out_shape=jax.ShapeDtypeStruct((V, D), table.dtype),
        grid=(V // blk,),
        in_specs=[pl.BlockSpec((blk, D), lambda i: (i, 0))],
        out_specs=pl.BlockSpec((blk, D), lambda i: (i, 0)),
    )(table)


@functools.cache
def _make_gather(B, D):
    b_per_w = B // _NW
    n_chunks = b_per_w // _CHUNK
    assert n_chunks % 2 == 0
    mesh = plsc.VectorSubcoreMesh(core_axis_name="c", subcore_axis_name="s")

    @functools.partial(
        pl.kernel,
        mesh=mesh,
        out_type=jax.ShapeDtypeStruct((B, D), jnp.float32),
        scratch_types=[
            pltpu.VMEM((b_per_w,), jnp.int32),
            pltpu.VMEM((2, _CHUNK, D), jnp.float32),
            pltpu.SemaphoreType.DMA,
            pltpu.SemaphoreType.DMA,
            pltpu.SemaphoreType.DMA,
        ],
    )
    def k(table_hbm, idx_hbm, out_hbm, idx_v, rows_v, gsem, osem0, osem1):
        wid = lax.axis_index("s") * _NC + lax.axis_index("c")
        base = pl.multiple_of(wid * b_per_w, 8)
        pltpu.sync_copy(idx_hbm.at[pl.ds(base, b_per_w)], idx_v)

        osem = (osem0, osem1)

        def gather(c, slot):
            off = pl.multiple_of(c * _CHUNK, 8)
            pltpu.async_copy(
                table_hbm.at[idx_v.at[pl.ds(off, _CHUNK)]],
                rows_v.at[slot], gsem)

        def put(c, slot):
            off = pl.multiple_of(base + c * _CHUNK, 8)
            pltpu.async_copy(
                rows_v.at[slot], out_hbm.at[pl.ds(off, _CHUNK)], osem[slot])

        def wait_g(slot):
            pltpu.make_async_copy(
                table_hbm.at[idx_v.at[pl.ds(0, _CHUNK)]],
                rows_v.at[slot], gsem).wait()

        def wait_p(slot):
            pltpu.make_async_copy(
                rows_v.at[slot], out_hbm.at[pl.ds(base, _CHUNK)],
                osem[slot]).wait()

        gather(0, 0)
        n_pairs = n_chunks // 2

        def body(i, carry):
            c0 = i * 2
            wait_g(0)

            @pl.when(i >= 1)
            def _():
                wait_p(1)
            gather(c0 + 1, 1)
            put(c0, 0)
            wait_g(1)

            @pl.when(i < n_pairs - 1)
            def _():
                wait_p(0)
                gather(c0 + 2, 0)
            put(c0 + 1, 1)
            return carry

        lax.fori_loop(0, n_pairs, body, 0)
        wait_p(0)
        wait_p(1)

    return k


def kernel(x, table):
    B = x.size
    flat = x.reshape(B)
    table2 = _scaled_table(table)
    out = _make_gather(B, D_MODEL)(table2, flat)
    return out.reshape(x.shape + (D_MODEL,))


# speedup vs baseline: 9.1672x; 1.3723x over previous
import functools
import math

import jax
import jax.numpy as jnp
from jax import lax
from jax.experimental import pallas as pl
from jax.experimental.pallas import tpu as pltpu
from jax.experimental.pallas import tpu_sc as plsc

D_MODEL = 128
SQRT_D = jnp.float32(math.sqrt(D_MODEL))

_NC = 2
_NS = 16
_NW = _NC * _NS
_CHUNK = 128
_LANES = 16
_NBUF = 4


@functools.cache
def _make_gather(B, D):
    b_per_w = B // _NW
    n_chunks = b_per_w // _CHUNK
    assert n_chunks % _NBUF == 0 and n_chunks >= 2 * _NBUF
    mesh = plsc.VectorSubcoreMesh(core_axis_name="c", subcore_axis_name="s")

    @functools.partial(
        pl.kernel,
        mesh=mesh,
        out_type=jax.ShapeDtypeStruct((B, D), jnp.float32),
        scratch_types=[
            pltpu.VMEM((b_per_w,), jnp.int32),
            pltpu.VMEM((_NBUF, _CHUNK, D), jnp.float32),
        ] + [pltpu.SemaphoreType.DMA] * (2 * _NBUF),
    )
    def k(table_hbm, idx_hbm, out_hbm, idx_v, rows_v, *sems):
        gsem = sems[:_NBUF]
        osem = sems[_NBUF:]
        wid = lax.axis_index("s") * _NC + lax.axis_index("c")
        base = pl.multiple_of(wid * b_per_w, 8)
        pltpu.sync_copy(idx_hbm.at[pl.ds(base, b_per_w)], idx_v)

        def gather(c, slot):
            off = pl.multiple_of(c * _CHUNK, 8)
            pltpu.async_copy(
                table_hbm.at[idx_v.at[pl.ds(off, _CHUNK)]],
                rows_v.at[slot], gsem[slot])

        def put(c, slot):
            off = pl.multiple_of(base + c * _CHUNK, 8)
            pltpu.async_copy(
                rows_v.at[slot], out_hbm.at[pl.ds(off, _CHUNK)], osem[slot])

        def wait_g(slot):
            pltpu.make_async_copy(
                table_hbm.at[idx_v.at[pl.ds(0, _CHUNK)]],
                rows_v.at[slot], gsem[slot]).wait()

        def wait_p(slot):
            pltpu.make_async_copy(
                rows_v.at[slot], out_hbm.at[pl.ds(base, _CHUNK)],
                osem[slot]).wait()

        def add_sqrt(slot):
            @plsc.parallel_loop(0, _CHUNK, unroll=4)
            def _(r):
                for j in range(D // _LANES):
                    sl = pl.ds(j * _LANES, _LANES)
                    rows_v[slot, r, sl] = rows_v[slot, r, sl] + SQRT_D

        for c in range(_NBUF - 1):
            gather(c, c)

        n_groups = n_chunks // _NBUF

        def body(g, carry):
            c0 = g * _NBUF
            for k_ in range(_NBUF):
                c = c0 + k_
                slot = k_
                nxt = (k_ + _NBUF - 1) % _NBUF
                wait_g(slot)
                add_sqrt(slot)
                put(c, slot)
                if k_ == 0:
                    @pl.when(g >= 1)
                    def _():
                        wait_p(nxt)
                        gather(c + _NBUF - 1, nxt)

                    @pl.when(g == 0)
                    def _():
                        gather(c + _NBUF - 1, nxt)
                else:
                    @pl.when(c + _NBUF - 1 < n_chunks)
                    def _():
                        wait_p(nxt)
                        gather(c + _NBUF - 1, nxt)
            return carry

        lax.fori_loop(0, n_groups, body, 0)
        for s in range(_NBUF):
            wait_p(s)

    return k


def kernel(x, table):
    B = x.size
    flat = x.reshape(B)
    out = _make_gather(B, D_MODEL)(table, flat)
    return out.reshape(x.shape + (D_MODEL,))


# speedup vs baseline: 9.1714x; 1.0005x over previous
import functools
import math

import jax
import jax.numpy as jnp
from jax import lax
from jax.experimental import pallas as pl
from jax.experimental.pallas import tpu as pltpu
from jax.experimental.pallas import tpu_sc as plsc

D_MODEL = 128
SQRT_D = jnp.float32(math.sqrt(D_MODEL))

_NC = 2
_NS = 16
_NW = _NC * _NS
_CHUNK = 128
_LANES = 16
_NBUF = 5


@functools.cache
def _make_gather(B, D):
    b_per_w = B // _NW
    n_chunks = b_per_w // _CHUNK
    assert n_chunks % _NBUF == 0 and n_chunks >= 2 * _NBUF
    mesh = plsc.VectorSubcoreMesh(core_axis_name="c", subcore_axis_name="s")

    @functools.partial(
        pl.kernel,
        mesh=mesh,
        out_type=jax.ShapeDtypeStruct((B, D), jnp.float32),
        scratch_types=[
            pltpu.VMEM((b_per_w,), jnp.int32),
            pltpu.VMEM((_NBUF, _CHUNK, D), jnp.float32),
        ] + [pltpu.SemaphoreType.DMA] * (2 * _NBUF),
    )
    def k(table_hbm, idx_hbm, out_hbm, idx_v, rows_v, *sems):
        gsem = sems[:_NBUF]
        osem = sems[_NBUF:]
        wid = lax.axis_index("s") * _NC + lax.axis_index("c")
        base = pl.multiple_of(wid * b_per_w, 8)
        pltpu.sync_copy(idx_hbm.at[pl.ds(base, b_per_w)], idx_v)

        def gather(c, slot):
            off = pl.multiple_of(c * _CHUNK, 8)
            pltpu.async_copy(
                table_hbm.at[idx_v.at[pl.ds(off, _CHUNK)]],
                rows_v.at[slot], gsem[slot])

        def put(c, slot):
            off = pl.multiple_of(base + c * _CHUNK, 8)
            pltpu.async_copy(
                rows_v.at[slot], out_hbm.at[pl.ds(off, _CHUNK)], osem[slot])

        def wait_g(slot):
            pltpu.make_async_copy(
                table_hbm.at[idx_v.at[pl.ds(0, _CHUNK)]],
                rows_v.at[slot], gsem[slot]).wait()

        def wait_p(slot):
            pltpu.make_async_copy(
                rows_v.at[slot], out_hbm.at[pl.ds(base, _CHUNK)],
                osem[slot]).wait()

        def add_sqrt(slot):
            @plsc.parallel_loop(0, _CHUNK, unroll=4)
            def _(r):
                for j in range(D // _LANES):
                    sl = pl.ds(j * _LANES, _LANES)
                    rows_v[slot, r, sl] = rows_v[slot, r, sl] + SQRT_D

        for c in range(_NBUF - 1):
            gather(c, c)

        n_groups = n_chunks // _NBUF

        def body(g, carry):
            c0 = g * _NBUF
            for k_ in range(_NBUF):
                c = c0 + k_
                slot = k_
                nxt = (k_ + _NBUF - 1) % _NBUF
                wait_g(slot)
                add_sqrt(slot)
                put(c, slot)
                if k_ == 0:
                    @pl.when(g >= 1)
                    def _():
                        wait_p(nxt)
                        gather(c + _NBUF - 1, nxt)

                    @pl.when(g == 0)
                    def _():
                        gather(c + _NBUF - 1, nxt)
                else:
                    @pl.when(c + _NBUF - 1 < n_chunks)
                    def _():
                        wait_p(nxt)
                        gather(c + _NBUF - 1, nxt)
            return carry

        lax.fori_loop(0, n_groups, body, 0)
        for s in range(_NBUF):
            wait_p(s)

    return k


def kernel(x, table):
    B = x.size
    flat = x.reshape(B)
    out = _make_gather(B, D_MODEL)(table, flat)
    return out.reshape(x.shape + (D_MODEL,))


# speedup vs baseline: 9.1940x; 1.0025x over previous
import functools
import math

import jax
import jax.numpy as jnp
from jax import lax
from jax.experimental import pallas as pl
from jax.experimental.pallas import tpu as pltpu
from jax.experimental.pallas import tpu_sc as plsc

D_MODEL = 128
SQRT_D = jnp.float32(math.sqrt(D_MODEL))

_NC = 2
_NS = 16
_NW = _NC * _NS
_CHUNK = 64
_LANES = 16
_NBUF = 8


@functools.cache
def _make_gather(B, D):
    b_per_w = B // _NW
    n_chunks = b_per_w // _CHUNK
    assert n_chunks % _NBUF == 0 and n_chunks >= 2 * _NBUF
    mesh = plsc.VectorSubcoreMesh(core_axis_name="c", subcore_axis_name="s")

    @functools.partial(
        pl.kernel,
        mesh=mesh,
        out_type=jax.ShapeDtypeStruct((B, D), jnp.float32),
        scratch_types=[
            pltpu.VMEM((b_per_w,), jnp.int32),
            pltpu.VMEM((_NBUF, _CHUNK, D), jnp.float32),
        ] + [pltpu.SemaphoreType.DMA] * (2 * _NBUF),
    )
    def k(table_hbm, idx_hbm, out_hbm, idx_v, rows_v, *sems):
        gsem = sems[:_NBUF]
        osem = sems[_NBUF:]
        wid = lax.axis_index("s") * _NC + lax.axis_index("c")
        base = pl.multiple_of(wid * b_per_w, 8)
        pltpu.sync_copy(idx_hbm.at[pl.ds(base, b_per_w)], idx_v)

        def gather(c, slot):
            off = pl.multiple_of(c * _CHUNK, 8)
            pltpu.async_copy(
                table_hbm.at[idx_v.at[pl.ds(off, _CHUNK)]],
                rows_v.at[slot], gsem[slot])

        def put(c, slot):
            off = pl.multiple_of(base + c * _CHUNK, 8)
            pltpu.async_copy(
                rows_v.at[slot], out_hbm.at[pl.ds(off, _CHUNK)], osem[slot])

        def wait_g(slot):
            pltpu.make_async_copy(
                table_hbm.at[idx_v.at[pl.ds(0, _CHUNK)]],
                rows_v.at[slot], gsem[slot]).wait()

        def wait_p(slot):
            pltpu.make_async_copy(
                rows_v.at[slot], out_hbm.at[pl.ds(base, _CHUNK)],
                osem[slot]).wait()

        def add_sqrt(slot):
            @plsc.parallel_loop(0, _CHUNK, unroll=4)
            def _(r):
                for j in range(D // _LANES):
                    sl = pl.ds(j * _LANES, _LANES)
                    rows_v[slot, r, sl] = rows_v[slot, r, sl] + SQRT_D

        for c in range(_NBUF - 1):
            gather(c, c)

        n_groups = n_chunks // _NBUF

        def body(g, carry):
            c0 = g * _NBUF
            for k_ in range(_NBUF):
                c = c0 + k_
                slot = k_
                nxt = (k_ + _NBUF - 1) % _NBUF
                wait_g(slot)
                add_sqrt(slot)
                put(c, slot)
                if k_ == 0:
                    @pl.when(g >= 1)
                    def _():
                        wait_p(nxt)
                        gather(c + _NBUF - 1, nxt)

                    @pl.when(g == 0)
                    def _():
                        gather(c + _NBUF - 1, nxt)
                else:
                    @pl.when(c + _NBUF - 1 < n_chunks)
                    def _():
                        wait_p(nxt)
                        gather(c + _NBUF - 1, nxt)
            return carry

        lax.fori_loop(0, n_groups, body, 0)
        for s in range(_NBUF):
            wait_p(s)

    return k


def kernel(x, table):
    B = x.size
    flat = x.reshape(B)
    out = _make_gather(B, D_MODEL)(table, flat)
    return out.reshape(x.shape + (D_MODEL,))


# speedup vs baseline: 9.2030x; 1.0010x over previous
import functools
import math

import jax
import jax.numpy as jnp
from jax import lax
from jax.experimental import pallas as pl
from jax.experimental.pallas import tpu as pltpu
from jax.experimental.pallas import tpu_sc as plsc

D_MODEL = 128
SQRT_D = jnp.float32(math.sqrt(D_MODEL))

_NC = 2
_NS = 16
_NW = _NC * _NS
_CHUNK = 64
_LANES = 16
_NBUF = 8


@functools.cache
def _make_gather(B, D):
    b_per_w = B // _NW
    n_chunks = b_per_w // _CHUNK
    assert n_chunks % _NBUF == 0 and n_chunks >= 2 * _NBUF
    mesh = plsc.VectorSubcoreMesh(core_axis_name="c", subcore_axis_name="s")

    @functools.partial(
        pl.kernel,
        mesh=mesh,
        out_type=jax.ShapeDtypeStruct((B, D), jnp.float32),
        scratch_types=[
            pltpu.VMEM((b_per_w,), jnp.int32),
            pltpu.VMEM((_NBUF, _CHUNK, D), jnp.float32),
        ] + [pltpu.SemaphoreType.DMA] * (2 * _NBUF),
    )
    def k(table_hbm, idx_hbm, out_hbm, idx_v, rows_v, *sems):
        gsem = sems[:_NBUF]
        osem = sems[_NBUF:]
        wid = lax.axis_index("s") * _NC + lax.axis_index("c")
        base = pl.multiple_of(wid * b_per_w, 8)
        pltpu.sync_copy(idx_hbm.at[pl.ds(base, b_per_w)], idx_v)

        def gather(c, slot):
            off = pl.multiple_of(c * _CHUNK, 8)
            pltpu.async_copy(
                table_hbm.at[idx_v.at[pl.ds(off, _CHUNK)]],
                rows_v.at[slot], gsem[slot])

        def put(c, slot):
            off = pl.multiple_of(base + c * _CHUNK, 8)
            pltpu.async_copy(
                rows_v.at[slot], out_hbm.at[pl.ds(off, _CHUNK)], osem[slot])

        def wait_g(slot):
            pltpu.make_async_copy(
                table_hbm.at[idx_v.at[pl.ds(0, _CHUNK)]],
                rows_v.at[slot], gsem[slot]).wait()

        def wait_p(slot):
            pltpu.make_async_copy(
                rows_v.at[slot], out_hbm.at[pl.ds(base, _CHUNK)],
                osem[slot]).wait()

        def add_sqrt(slot):
            @plsc.parallel_loop(0, _CHUNK, unroll=4)
            def _(r):
                for j in range(D // _LANES):
                    sl = pl.ds(j * _LANES, _LANES)
                    rows_v[slot, r, sl] = rows_v[slot, r, sl] + SQRT_D

        for c in range(_NBUF - 1):
            gather(c, c)

        n_groups = n_chunks // _NBUF

        def body(g, carry):
            c0 = g * _NBUF
            for k_ in range(_NBUF):
                c = c0 + k_
                slot = k_
                nxt = (k_ + _NBUF - 1) % _NBUF
                wait_g(slot)
                add_sqrt(slot)
                put(c, slot)
                if k_ == 0:
                    @pl.when(g >= 1)
                    def _():
                        wait_p(nxt)
                        gather(c + _NBUF - 1, nxt)

                    @pl.when(g == 0)
                    def _():
                        gather(c + _NBUF - 1, nxt)
                else:
                    @pl.when(c + _NBUF - 1 < n_chunks)
                    def _():
                        wait_p(nxt)
                        gather(c + _NBUF - 1, nxt)
            return carry

        lax.fori_loop(0, n_groups, body, 0)
        for s in range(_NBUF):
            wait_p(s)

    return k


def kernel(x, table):
    B = x.size
    flat = x.reshape(B).astype(jnp.int32)
    out = _make_gather(B, D_MODEL)(table, flat)
    return out.reshape(x.shape + (D_MODEL,))
